# baseline (speedup 1.0000x reference)
import jax
import jax.numpy as jnp
from jax import lax
from jax.experimental import pallas as pl
from jax.experimental.pallas import tpu as pltpu
from jax.experimental.pallas import tpu_sc as plsc

NUM_WORKERS = 32
CHUNK_ROWS = 4
LANES = 16


def _build(B, S, D, n_table):
    assert B % NUM_WORKERS == 0
    rows_per_w = B // NUM_WORKERS
    assert rows_per_w % (2 * CHUNK_ROWS) == 0
    n_chunks = rows_per_w // CHUNK_ROWS
    CS = CHUNK_ROWS * S
    assert D % LANES == 0
    d_regs = D // LANES
    assert S % 4 == 0
    W = S // 4
    n_wgroups = W // LANES
    wtail_words = W - n_wgroups * LANES
    wtail_off = W - LANES
    row_slices = []
    off = 0
    while off < S:
        n = min(128, S - off)
        n -= n % 8
        row_slices.append((off, n))
        off += n
    assert off == S

    mesh = plsc.VectorSubcoreMesh(core_axis_name="c", subcore_axis_name="s")

    def body(x_hbm, mw_hbm, table_hbm, out_hbm,
             xv0, mw0, rows0, xv1, mw1, rows1, outb, gsem0, gsem1):
        wid = lax.axis_index("s") * 2 + lax.axis_index("c")
        bufs = ((xv0, mw0, rows0, gsem0), (xv1, mw1, rows1, gsem1))

        def load_idx(c, buf):
            xv, mwv, _, _ = buf
            row0 = wid * rows_per_w + c * CHUNK_ROWS
            pltpu.sync_copy(x_hbm.at[pl.ds(row0, CHUNK_ROWS), :], xv)
            pltpu.sync_copy(mw_hbm.at[pl.ds(row0, CHUNK_ROWS), :], mwv)

        def gather_copies(buf):
            xv, _, rows_v, gsem = buf
            for r in range(CHUNK_ROWS):
                for off, n in row_slices:
                    yield pltpu.make_async_copy(
                        table_hbm.at[xv.at[r, pl.ds(off, n)]],
                        rows_v.at[pl.ds(r * S + off, n)],
                        gsem,
                    )

        def fire(buf):
            for cp in gather_copies(buf):
                cp.start()

        def wait(buf):
            for cp in gather_copies(buf):
                cp.wait()

        def process(c, buf):
            _, mwv, rows_v, _ = buf
            row0 = wid * rows_per_w + c * CHUNK_ROWS

            def row_body(r, carry):
                rb = r * S

                def accum_words(base, wvec, jlanes, accs, cnt):
                    out = list(accs)
                    for jj, j in enumerate(jlanes):
                        word = wvec[j]
                        for bpos in range(4):
                            row = base + jj * 4 + bpos
                            if bpos:
                                bit = lax.shift_right_logical(
                                    word, jnp.int32(8 * bpos)
                                ) & jnp.int32(1)
                            else:
                                bit = word & jnp.int32(1)
                            cnt = cnt + bit
                            m = jnp.full(
                                (LANES,), bit.astype(jnp.float32), jnp.float32
                            )
                            for d in range(d_regs):
                                out[d] = (
                                    out[d]
                                    + rows_v[row, pl.ds(d * LANES, LANES)] * m
                                )
                    return tuple(out), cnt

                def wg_body(wg, ac):
                    accs, cnt = ac
                    wvec = mwv[r, pl.ds(wg * LANES, LANES)]
                    base = rb + wg * (LANES * 4)
                    return accum_words(base, wvec, range(LANES), accs, cnt)

                z = jnp.zeros((LANES,), jnp.float32)
                accs, cnt = lax.fori_loop(
                    0, n_wgroups, wg_body, ((z,) * d_regs, jnp.int32(0))
                )
                if wtail_words:
                    wvec = mwv[r, pl.ds(wtail_off, LANES)]
                    base = rb + n_wgroups * (LANES * 4)
                    accs, cnt = accum_words(
                        base,
                        wvec,
                        range(LANES - wtail_words, LANES),
                        accs,
                        cnt,
                    )
                inv = jnp.float32(1.0) / (
                    jnp.full((LANES,), cnt.astype(jnp.float32), jnp.float32)
                    + jnp.float32(1e-9)
                )
                for d in range(d_regs):
                    outb[r, pl.ds(d * LANES, LANES)] = accs[d] * inv
                return carry

            lax.fori_loop(0, CHUNK_ROWS, row_body, 0)
            pltpu.sync_copy(outb, out_hbm.at[pl.ds(row0, CHUNK_ROWS)])

        load_idx(0, bufs[0])
        fire(bufs[0])

        def pair_body(i, carry):
            c0 = 2 * i
            load_idx(c0 + 1, bufs[1])
            fire(bufs[1])
            wait(bufs[0])
            process(c0, bufs[0])

            @pl.when(c0 + 2 < n_chunks)
            def _():
                load_idx(c0 + 2, bufs[0])
                fire(bufs[0])

            wait(bufs[1])
            process(c0 + 1, bufs[1])
            return carry

        lax.fori_loop(0, n_chunks // 2, pair_body, 0)

    return pl.kernel(
        body,
        out_type=jax.ShapeDtypeStruct((B, D), jnp.float32),
        mesh=mesh,
        compiler_params=pltpu.CompilerParams(use_tc_tiling_on_sc=False),
        scratch_types=[
            pltpu.VMEM((CHUNK_ROWS, S), jnp.int32),
            pltpu.VMEM((CHUNK_ROWS, W), jnp.int32),
            pltpu.VMEM((CS, D), jnp.float32),
            pltpu.VMEM((CHUNK_ROWS, S), jnp.int32),
            pltpu.VMEM((CHUNK_ROWS, W), jnp.int32),
            pltpu.VMEM((CS, D), jnp.float32),
            pltpu.VMEM((CHUNK_ROWS, D), jnp.float32),
            pltpu.SemaphoreType.DMA,
            pltpu.SemaphoreType.DMA,
        ],
    )


@jax.jit
def kernel(x, mask, table):
    B, S = x.shape
    n_table, D = table.shape
    mw = lax.bitcast_convert_type(
        mask.view(jnp.int8).reshape(B, S // 4, 4), jnp.int32
    )
    return _build(B, S, D, n_table)(x.astype(jnp.int32), mw, table)
